# Initial kernel scaffold; baseline (speedup 1.0000x reference)
import jax
import jax.numpy as jnp
from jax.experimental import pallas as pl


def kernel(x, edge_index, W1, b1, g1, be1, W2, b2, g2, be2, W3, b3):
    raise NotImplementedError("write your pallas kernel here")


# speedup vs baseline: 3.4208x; 3.4208x over previous
import functools

import jax
import jax.numpy as jnp
from jax import lax
from jax.experimental import pallas as pl
from jax.experimental.pallas import tpu as pltpu
from jax.experimental.pallas import tpu_sc as plsc

_NC = 2
_NS = 16
_K = 128


def _round_up(v, m):
    return (v + m - 1) // m * m


def _deg_body(ept, rpt, src_ref, dst_ref, out_ref, idx_s, idx_d, ones_v, zbuf,
              deg_o, deg_i):
    c = lax.axis_index("c")
    s = lax.axis_index("s")
    wid = c * _NS + s

    for j in range(8):
        ones_v[pl.ds(j * 16, 16)] = jnp.ones((16,), jnp.float32)

    def zloop(i, carry):
        zbuf[pl.ds(i * 16, 16)] = jnp.zeros((16,), jnp.float32)
        return carry

    lax.fori_loop(0, rpt // 16, zloop, 0)
    pltpu.sync_copy(zbuf, deg_o.at[pl.ds(s * rpt, rpt)])
    pltpu.sync_copy(zbuf, deg_i.at[pl.ds(s * rpt, rpt)])
    plsc.subcore_barrier()

    def eloop(k, carry):
        base = wid * ept + k * _K
        pltpu.sync_copy(src_ref.at[pl.ds(base, _K)], idx_s)
        pltpu.sync_copy(dst_ref.at[pl.ds(base, _K)], idx_d)
        pltpu.sync_copy(ones_v, deg_o.at[idx_s], add=True)
        pltpu.sync_copy(ones_v, deg_i.at[idx_d], add=True)
        return carry

    lax.fori_loop(0, ept // _K, eloop, 0)
    plsc.subcore_barrier()
    pltpu.sync_copy(deg_o.at[pl.ds(s * rpt, rpt)],
                    out_ref.at[c, 0, pl.ds(s * rpt, rpt)])
    pltpu.sync_copy(deg_i.at[pl.ds(s * rpt, rpt)],
                    out_ref.at[c, 1, pl.ds(s * rpt, rpt)])


def _agg_body(ept, rpt, h_ref, src_ref, dst_ref, out_ref, idx_s, idx_d, rows,
              zbuf, agg, sem):
    c = lax.axis_index("c")
    s = lax.axis_index("s")
    wid = c * _NS + s

    def zloop(i, carry):
        zbuf[i // 8, pl.ds((i % 8) * 16, 16)] = jnp.zeros((16,), jnp.float32)
        return carry

    lax.fori_loop(0, 128 * 8, zloop, 0)
    for j in range(rpt // 128):
        pltpu.sync_copy(zbuf, agg.at[pl.ds(s * rpt + j * 128, 128)])
    plsc.subcore_barrier()

    def eloop(k, carry):
        base = wid * ept + k * _K
        pltpu.sync_copy(src_ref.at[pl.ds(base, _K)], idx_s)
        pltpu.sync_copy(dst_ref.at[pl.ds(base, _K)], idx_d)
        pltpu.async_copy(h_ref.at[idx_s], rows, sem).wait()
        pltpu.sync_copy(rows, agg.at[idx_d], add=True)
        return carry

    lax.fori_loop(0, ept // _K, eloop, 0)
    plsc.subcore_barrier()
    pltpu.sync_copy(agg.at[pl.ds(s * rpt, rpt)],
                    out_ref.at[c, pl.ds(s * rpt, rpt)])


@functools.lru_cache(maxsize=None)
def _make_deg_kernel(ep, np_):
    ept = ep // (_NC * _NS)
    rpt = np_ // _NS
    mesh = plsc.VectorSubcoreMesh(core_axis_name="c", subcore_axis_name="s")
    return pl.kernel(
        functools.partial(_deg_body, ept, rpt),
        out_type=jax.ShapeDtypeStruct((_NC, 2, np_), jnp.float32),
        mesh=mesh,
        scratch_types=[
            pltpu.VMEM((_K,), jnp.int32),
            pltpu.VMEM((_K,), jnp.int32),
            pltpu.VMEM((_K,), jnp.float32),
            pltpu.VMEM((rpt,), jnp.float32),
            pltpu.VMEM_SHARED((np_,), jnp.float32),
            pltpu.VMEM_SHARED((np_,), jnp.float32),
        ],
    )


@functools.lru_cache(maxsize=None)
def _make_agg_kernel(ep, np_, d):
    ept = ep // (_NC * _NS)
    rpt = np_ // _NS
    mesh = plsc.VectorSubcoreMesh(core_axis_name="c", subcore_axis_name="s")
    return pl.kernel(
        functools.partial(_agg_body, ept, rpt),
        out_type=jax.ShapeDtypeStruct((_NC, np_, d), jnp.float32),
        mesh=mesh,
        scratch_types=[
            pltpu.VMEM((_K,), jnp.int32),
            pltpu.VMEM((_K,), jnp.int32),
            pltpu.VMEM((_K, d), jnp.float32),
            pltpu.VMEM((128, d), jnp.float32),
            pltpu.VMEM_SHARED((np_, d), jnp.float32),
            pltpu.SemaphoreType.DMA,
        ],
    )


def _prep_tc_body(x_ref, dinv_ref, out_ref):
    out_ref[...] = x_ref[...] * dinv_ref[...]


def _layer_tc_body(p_ref, di_ref, do_ref, w_ref, b_ref, g_ref, be_ref,
                   out_ref):
    t = (p_ref[0] + p_ref[1]) * di_ref[...]
    y = jnp.dot(t, w_ref[...], preferred_element_type=jnp.float32)
    y = y + b_ref[...]
    e = jnp.where(y > 0, y, jnp.exp(jnp.minimum(y, 0.0)) - 1.0)
    mu = jnp.mean(e, axis=-1, keepdims=True)
    d = e - mu
    var = jnp.mean(d * d, axis=-1, keepdims=True)
    ln = d * lax.rsqrt(var + 1e-5) * g_ref[...] + be_ref[...]
    out_ref[...] = ln * do_ref[...]


def _final_tc_body(p_ref, di_ref, w_ref, b_ref, out_ref):
    t = (p_ref[0] + p_ref[1]) * di_ref[...]
    y = jnp.dot(t, w_ref[...], preferred_element_type=jnp.float32)
    out_ref[...] = y + b_ref[...]


def _prep_tc(x_p, dinv_o):
    np_, d = x_p.shape
    blk = 512
    return pl.pallas_call(
        _prep_tc_body,
        grid=(np_ // blk,),
        in_specs=[
            pl.BlockSpec((blk, d), lambda i: (i, 0)),
            pl.BlockSpec((blk, 1), lambda i: (i, 0)),
        ],
        out_specs=pl.BlockSpec((blk, d), lambda i: (i, 0)),
        out_shape=jax.ShapeDtypeStruct((np_, d), jnp.float32),
    )(x_p, dinv_o)


def _layer_tc(p, dinv_i, dinv_o, w, b, g, be):
    _, np_, d = p.shape
    h = w.shape[1]
    blk = 512
    return pl.pallas_call(
        _layer_tc_body,
        grid=(np_ // blk,),
        in_specs=[
            pl.BlockSpec((_NC, blk, d), lambda i: (0, i, 0)),
            pl.BlockSpec((blk, 1), lambda i: (i, 0)),
            pl.BlockSpec((blk, 1), lambda i: (i, 0)),
            pl.BlockSpec((d, h), lambda i: (0, 0)),
            pl.BlockSpec((1, h), lambda i: (0, 0)),
            pl.BlockSpec((1, h), lambda i: (0, 0)),
            pl.BlockSpec((1, h), lambda i: (0, 0)),
        ],
        out_specs=pl.BlockSpec((blk, h), lambda i: (i, 0)),
        out_shape=jax.ShapeDtypeStruct((np_, h), jnp.float32),
    )(p, dinv_i, dinv_o, w, b, g, be)


def _final_tc(p, dinv_i, w, b):
    _, np_, d = p.shape
    h = w.shape[1]
    blk = 512
    return pl.pallas_call(
        _final_tc_body,
        grid=(np_ // blk,),
        in_specs=[
            pl.BlockSpec((_NC, blk, d), lambda i: (0, i, 0)),
            pl.BlockSpec((blk, 1), lambda i: (i, 0)),
            pl.BlockSpec((d, h), lambda i: (0, 0)),
            pl.BlockSpec((1, h), lambda i: (0, 0)),
        ],
        out_specs=pl.BlockSpec((blk, h), lambda i: (i, 0)),
        out_shape=jax.ShapeDtypeStruct((np_, h), jnp.float32),
    )(p, dinv_i, w, b)


def kernel(x, edge_index, W1, b1, g1, be1, W2, b2, g2, be2, W3, b3):
    n, d = x.shape
    e = edge_index.shape[1]
    c = W3.shape[1]
    np_ = _NS * _round_up(-(-n // _NS), 128)
    ep = _round_up(e, _NC * _NS * _K)
    dump = np_ - 1

    src = jnp.pad(edge_index[0], (0, ep - e), constant_values=dump)
    dst = jnp.pad(edge_index[1], (0, ep - e), constant_values=dump)
    x_p = jnp.pad(x, ((0, np_ - n), (0, 0)))

    degp = _make_deg_kernel(ep, np_)(src, dst)
    deg_o = jnp.maximum(degp[0, 0] + degp[1, 0], 1.0)
    deg_i = jnp.maximum(degp[0, 1] + degp[1, 1], 1.0)
    dinv_o = lax.rsqrt(deg_o).reshape(np_, 1)
    dinv_i = lax.rsqrt(deg_i).reshape(np_, 1)

    agg = _make_agg_kernel(ep, np_, d)

    hs = _prep_tc(x_p, dinv_o)
    p = agg(hs, src, dst)
    hs = _layer_tc(p, dinv_i, dinv_o, W1, b1.reshape(1, -1),
                   g1.reshape(1, -1), be1.reshape(1, -1))
    p = agg(hs, src, dst)
    hs = _layer_tc(p, dinv_i, dinv_o, W2, b2.reshape(1, -1),
                   g2.reshape(1, -1), be2.reshape(1, -1))
    p = agg(hs, src, dst)

    w3p = jnp.pad(W3, ((0, 0), (0, d - c)))
    b3p = jnp.pad(b3, (0, d - c)).reshape(1, -1)
    out = _final_tc(p, dinv_i, w3p, b3p)
    return out[:n, :c]


# speedup vs baseline: 3.7068x; 1.0836x over previous
import functools

import jax
import jax.numpy as jnp
from jax import lax
from jax.experimental import pallas as pl
from jax.experimental.pallas import tpu as pltpu
from jax.experimental.pallas import tpu_sc as plsc

_NC = 2
_NS = 16
_K = 128


def _round_up(v, m):
    return (v + m - 1) // m * m


def _deg_body(cpt, rpt, src_ref, dst_ref, out_ref, idx_s, idx_d, ones_v, zbuf,
              deg_o, deg_i, sem):
    c = lax.axis_index("c")
    s = lax.axis_index("s")
    wid = c * _NS + s

    pltpu.sync_copy(src_ref.at[pl.ds(wid * cpt, cpt)], idx_s)
    pltpu.sync_copy(dst_ref.at[pl.ds(wid * cpt, cpt)], idx_d)
    for j in range(8):
        ones_v[pl.ds(j * 16, 16)] = jnp.ones((16,), jnp.float32)

    def zloop(i, carry):
        zbuf[pl.ds(i * 16, 16)] = jnp.zeros((16,), jnp.float32)
        return carry

    lax.fori_loop(0, rpt // 16, zloop, 0)
    pltpu.sync_copy(zbuf, deg_o.at[pl.ds(s * rpt, rpt)])
    pltpu.sync_copy(zbuf, deg_i.at[pl.ds(s * rpt, rpt)])
    plsc.subcore_barrier()

    def eloop(k, carry):
        pltpu.async_copy(ones_v, deg_o.at[idx_s.at[k]], sem, add=True)
        pltpu.async_copy(ones_v, deg_i.at[idx_d.at[k]], sem, add=True)
        return carry

    lax.fori_loop(0, cpt, eloop, 0)

    def dloop(k, carry):
        pltpu.make_async_copy(ones_v, deg_o.at[idx_s.at[k]], sem).wait()
        pltpu.make_async_copy(ones_v, deg_i.at[idx_d.at[k]], sem).wait()
        return carry

    lax.fori_loop(0, cpt, dloop, 0)
    plsc.subcore_barrier()
    pltpu.sync_copy(deg_o.at[pl.ds(s * rpt, rpt)],
                    out_ref.at[c, 0, pl.ds(s * rpt, rpt)])
    pltpu.sync_copy(deg_i.at[pl.ds(s * rpt, rpt)],
                    out_ref.at[c, 1, pl.ds(s * rpt, rpt)])


def _agg_body(cpt, rpt, h_ref, src_ref, dst_ref, out_ref, isr, idr,
              rows0, rows1, agg, is0, is1, is2, is3, gs0, gs1, ss0, ss1):
    c = lax.axis_index("c")
    s = lax.axis_index("s")
    wid = c * _NS + s
    base = wid * cpt
    rows = (rows0, rows1)
    gsem = (gs0, gs1)
    ssem = (ss0, ss1)
    isem = (is0, is1, is2, is3)

    def zloop(i, carry):
        rows0[i // 8, pl.ds((i % 8) * 16, 16)] = jnp.zeros((16,), jnp.float32)
        return carry

    lax.fori_loop(0, 128 * 8, zloop, 0)
    for j in range(rpt // 128):
        pltpu.sync_copy(rows0, agg.at[pl.ds(s * rpt + j * 128, 128)])

    for sl in range(4):
        pltpu.async_copy(src_ref.at[base + sl], isr.at[sl], isem[sl])
        pltpu.async_copy(dst_ref.at[base + sl], idr.at[sl], isem[sl])
    plsc.subcore_barrier()

    def eloop(gg, carry):
        for q in range(2):
            g = gg * 2 + q
            for b in range(2):
                k = g * 2 + b
                sl = 2 * q + b
                psl = (sl + 2) % 4

                @pl.when(g > 0)
                def _():
                    pltpu.make_async_copy(
                        rows[b], agg.at[idr.at[sl]], ssem[b]).wait()

                @pl.when(g < cpt // 2 - 1)
                def _():
                    pltpu.async_copy(src_ref.at[base + k + 2], isr.at[psl],
                                     isem[psl])
                    pltpu.async_copy(dst_ref.at[base + k + 2], idr.at[psl],
                                     isem[psl])

                pltpu.make_async_copy(src_ref.at[base + k], isr.at[sl],
                                      isem[sl]).wait()
                pltpu.make_async_copy(dst_ref.at[base + k], idr.at[sl],
                                      isem[sl]).wait()
                pltpu.async_copy(h_ref.at[isr.at[sl]], rows[b], gsem[b])
            for b in range(2):
                sl = 2 * q + b
                pltpu.make_async_copy(h_ref.at[isr.at[sl]], rows[b],
                                      gsem[b]).wait()
                pltpu.async_copy(rows[b], agg.at[idr.at[sl]], ssem[b],
                                 add=True)
        return carry

    lax.fori_loop(0, cpt // 4, eloop, 0)
    for b in range(2):
        pltpu.make_async_copy(rows[b], agg.at[idr.at[b]], ssem[b]).wait()
    plsc.subcore_barrier()
    pltpu.sync_copy(agg.at[pl.ds(s * rpt, rpt)],
                    out_ref.at[c, pl.ds(s * rpt, rpt)])


@functools.lru_cache(maxsize=None)
def _make_deg_kernel(ep, np_):
    cpt = ep // (_NC * _NS * _K)
    rpt = np_ // _NS
    mesh = plsc.VectorSubcoreMesh(core_axis_name="c", subcore_axis_name="s")
    return pl.kernel(
        functools.partial(_deg_body, cpt, rpt),
        out_type=jax.ShapeDtypeStruct((_NC, 2, np_), jnp.float32),
        mesh=mesh,
        scratch_types=[
            pltpu.VMEM((cpt, _K), jnp.int32),
            pltpu.VMEM((cpt, _K), jnp.int32),
            pltpu.VMEM((_K,), jnp.float32),
            pltpu.VMEM((rpt,), jnp.float32),
            pltpu.VMEM_SHARED((np_,), jnp.float32),
            pltpu.VMEM_SHARED((np_,), jnp.float32),
            pltpu.SemaphoreType.DMA,
        ],
    )


@functools.lru_cache(maxsize=None)
def _make_agg_kernel(ep, np_, d):
    cpt = ep // (_NC * _NS * _K)
    rpt = np_ // _NS
    mesh = plsc.VectorSubcoreMesh(core_axis_name="c", subcore_axis_name="s")
    return pl.kernel(
        functools.partial(_agg_body, cpt, rpt),
        out_type=jax.ShapeDtypeStruct((_NC, np_, d), jnp.float32),
        mesh=mesh,
        scratch_types=[
            pltpu.VMEM((4, _K), jnp.int32),
            pltpu.VMEM((4, _K), jnp.int32),
            pltpu.VMEM((_K, d), jnp.float32),
            pltpu.VMEM((_K, d), jnp.float32),
            pltpu.VMEM_SHARED((np_, d), jnp.float32),
        ] + [pltpu.SemaphoreType.DMA] * 8,
    )


def _prep_tc_body(x_ref, dinv_ref, out_ref):
    out_ref[...] = x_ref[...] * dinv_ref[...]


def _layer_tc_body(p_ref, di_ref, do_ref, w_ref, b_ref, g_ref, be_ref,
                   out_ref):
    t = (p_ref[0] + p_ref[1]) * di_ref[...]
    y = jnp.dot(t, w_ref[...], preferred_element_type=jnp.float32)
    y = y + b_ref[...]
    e = jnp.where(y > 0, y, jnp.exp(jnp.minimum(y, 0.0)) - 1.0)
    mu = jnp.mean(e, axis=-1, keepdims=True)
    d = e - mu
    var = jnp.mean(d * d, axis=-1, keepdims=True)
    ln = d * lax.rsqrt(var + 1e-5) * g_ref[...] + be_ref[...]
    out_ref[...] = ln * do_ref[...]


def _final_tc_body(p_ref, di_ref, w_ref, b_ref, out_ref):
    t = (p_ref[0] + p_ref[1]) * di_ref[...]
    y = jnp.dot(t, w_ref[...], preferred_element_type=jnp.float32)
    out_ref[...] = y + b_ref[...]


def _prep_tc(x_p, dinv_o):
    np_, d = x_p.shape
    blk = 512
    return pl.pallas_call(
        _prep_tc_body,
        grid=(np_ // blk,),
        in_specs=[
            pl.BlockSpec((blk, d), lambda i: (i, 0)),
            pl.BlockSpec((blk, 1), lambda i: (i, 0)),
        ],
        out_specs=pl.BlockSpec((blk, d), lambda i: (i, 0)),
        out_shape=jax.ShapeDtypeStruct((np_, d), jnp.float32),
    )(x_p, dinv_o)


def _layer_tc(p, dinv_i, dinv_o, w, b, g, be):
    _, np_, d = p.shape
    h = w.shape[1]
    blk = 512
    return pl.pallas_call(
        _layer_tc_body,
        grid=(np_ // blk,),
        in_specs=[
            pl.BlockSpec((_NC, blk, d), lambda i: (0, i, 0)),
            pl.BlockSpec((blk, 1), lambda i: (i, 0)),
            pl.BlockSpec((blk, 1), lambda i: (i, 0)),
            pl.BlockSpec((d, h), lambda i: (0, 0)),
            pl.BlockSpec((1, h), lambda i: (0, 0)),
            pl.BlockSpec((1, h), lambda i: (0, 0)),
            pl.BlockSpec((1, h), lambda i: (0, 0)),
        ],
        out_specs=pl.BlockSpec((blk, h), lambda i: (i, 0)),
        out_shape=jax.ShapeDtypeStruct((np_, h), jnp.float32),
    )(p, dinv_i, dinv_o, w, b, g, be)


def _final_tc(p, dinv_i, w, b):
    _, np_, d = p.shape
    h = w.shape[1]
    blk = 512
    return pl.pallas_call(
        _final_tc_body,
        grid=(np_ // blk,),
        in_specs=[
            pl.BlockSpec((_NC, blk, d), lambda i: (0, i, 0)),
            pl.BlockSpec((blk, 1), lambda i: (i, 0)),
            pl.BlockSpec((d, h), lambda i: (0, 0)),
            pl.BlockSpec((1, h), lambda i: (0, 0)),
        ],
        out_specs=pl.BlockSpec((blk, h), lambda i: (i, 0)),
        out_shape=jax.ShapeDtypeStruct((np_, h), jnp.float32),
    )(p, dinv_i, w, b)


def kernel(x, edge_index, W1, b1, g1, be1, W2, b2, g2, be2, W3, b3):
    n, d = x.shape
    e = edge_index.shape[1]
    c = W3.shape[1]
    np_ = _NS * _round_up(-(-n // _NS), 128)
    ep = _round_up(e, _NC * _NS * _K * 2)
    dump = np_ - 1

    src = jnp.pad(edge_index[0], (0, ep - e),
                  constant_values=dump).reshape(ep // _K, _K)
    dst = jnp.pad(edge_index[1], (0, ep - e),
                  constant_values=dump).reshape(ep // _K, _K)
    x_p = jnp.pad(x, ((0, np_ - n), (0, 0)))

    degp = _make_deg_kernel(ep, np_)(src, dst)
    deg_o = jnp.maximum(degp[0, 0] + degp[1, 0], 1.0)
    deg_i = jnp.maximum(degp[0, 1] + degp[1, 1], 1.0)
    dinv_o = lax.rsqrt(deg_o).reshape(np_, 1)
    dinv_i = lax.rsqrt(deg_i).reshape(np_, 1)

    agg = _make_agg_kernel(ep, np_, d)

    hs = _prep_tc(x_p, dinv_o)
    p = agg(hs, src, dst)
    hs = _layer_tc(p, dinv_i, dinv_o, W1, b1.reshape(1, -1),
                   g1.reshape(1, -1), be1.reshape(1, -1))
    p = agg(hs, src, dst)
    hs = _layer_tc(p, dinv_i, dinv_o, W2, b2.reshape(1, -1),
                   g2.reshape(1, -1), be2.reshape(1, -1))
    p = agg(hs, src, dst)

    w3p = jnp.pad(W3, ((0, 0), (0, d - c)))
    b3p = jnp.pad(b3, (0, d - c)).reshape(1, -1)
    out = _final_tc(p, dinv_i, w3p, b3p)
    return out[:n, :c]
